# Initial kernel scaffold; baseline (speedup 1.0000x reference)
import jax
import jax.numpy as jnp
from jax.experimental import pallas as pl


def kernel(features, points, proj_pts, invis_idx, height, width):
    raise NotImplementedError("write your pallas kernel here")


# speedup vs baseline: 1.3104x; 1.3104x over previous
import functools

import jax
import jax.numpy as jnp
from jax.experimental import pallas as pl
from jax.experimental.pallas import tpu as pltpu


def _normalize_body(f_ref, out_ref):
    f = f_ref[0]
    n2 = jnp.sum(f * f, axis=-1, keepdims=True)
    norm = jnp.maximum(jnp.sqrt(n2), 1e-8)
    out_ref[0] = (f / norm).astype(jnp.bfloat16)


def _match_body(fnj_ref, fni_ref, pts_ref, ppts_ref, t_ref, out_ref):
    i, j, r = pl.program_id(0), pl.program_id(1), pl.program_id(2)

    @pl.when((i == 0) & (j == 0) & (r == 0))
    def _init():
        out_ref[0, 0] = 0.0

    p = pts_ref[0]
    q = ppts_ref[0, 0]
    ab = jax.lax.dot_general(p, q, (((1,), (1,)), ((), ())),
                             preferred_element_type=jnp.float32)
    a2 = jnp.sum(p * p, axis=1)
    b2 = jnp.sum(q * q, axis=1)
    d2 = (a2[:, None] + b2[None, :] + t_ref[...]) - 2.0 * ab

    cos = jax.lax.dot_general(fnj_ref[0], fni_ref[0], (((1,), (1,)), ((), ())),
                              preferred_element_type=jnp.float32)
    contrib = jnp.sum(jnp.where(d2 <= 1.0, 1.0 - cos, 0.0))
    out_ref[0, 0] += contrib


def kernel(features, points, proj_pts, invis_idx, height, width):
    del invis_idx
    B, N, D = features.shape
    RB = 256
    nr = N // RB

    fn = pl.pallas_call(
        _normalize_body,
        grid=(B,),
        in_specs=[pl.BlockSpec((1, N, D), lambda b: (b, 0, 0))],
        out_specs=pl.BlockSpec((1, N, D), lambda b: (b, 0, 0)),
        out_shape=jax.ShapeDtypeStruct((B, N, D), jnp.bfloat16),
    )(features)

    factor = jnp.array([(width - 1.0) / 2.0, (height - 1.0) / 2.0],
                       jnp.float32)
    pts = (points + 1.0) * factor
    ppts = (proj_pts + 1.0) * factor
    row = jax.lax.broadcasted_iota(jnp.int32, (N, N), 0)
    col = jax.lax.broadcasted_iota(jnp.int32, (N, N), 1)
    tblk = jnp.where(row <= col, 0.0, 1e30).astype(jnp.float32)

    out = pl.pallas_call(
        _match_body,
        grid=(B, B, nr),
        in_specs=[
            pl.BlockSpec((1, RB, D), lambda i, j, r: (j, r, 0)),
            pl.BlockSpec((1, N, D), lambda i, j, r: (i, 0, 0)),
            pl.BlockSpec((1, RB, 2), lambda i, j, r: (i, r, 0)),
            pl.BlockSpec((1, 1, N, 2), lambda i, j, r: (i, j, 0, 0)),
            pl.BlockSpec((RB, N), lambda i, j, r: (r, 0)),
        ],
        out_specs=pl.BlockSpec((1, 1), lambda i, j, r: (0, 0),
                               memory_space=pltpu.SMEM),
        out_shape=jax.ShapeDtypeStruct((1, 1), jnp.float32),
        compiler_params=pltpu.CompilerParams(
            dimension_semantics=("arbitrary", "arbitrary", "arbitrary")),
    )(fn, fn, pts, ppts, tblk)
    return out[0, 0]


# speedup vs baseline: 1.4619x; 1.1156x over previous
import jax
import jax.numpy as jnp
from jax.experimental import pallas as pl
from jax.experimental.pallas import tpu as pltpu

_RB = 256


def _normalize_body(f_ref, out_ref):
    f = f_ref[0]
    n2 = jnp.sum(f * f, axis=-1, keepdims=True)
    norm = jnp.maximum(jnp.sqrt(n2), 1e-8)
    out_ref[0] = (f / norm).astype(jnp.bfloat16)


def _match_body(fnj_ref, fni_ref, pts_ref, ppts_ref, t_ref, out_ref):
    r, i, j = pl.program_id(0), pl.program_id(1), pl.program_id(2)

    @pl.when((i == 0) & (j == 0) & (r == 0))
    def _init():
        out_ref[0, 0] = 0.0

    p = pts_ref[0]
    q = ppts_ref[0, 0]
    ab = jax.lax.dot_general(p, q, (((1,), (1,)), ((), ())),
                             preferred_element_type=jnp.float32)
    a2h = 0.5 * jnp.sum(p * p, axis=1)
    b2h = 0.5 * jnp.sum(q * q, axis=1)
    d2h = (a2h[:, None] + b2h[None, :] + t_ref[...]) - ab
    cos = jax.lax.dot_general(fnj_ref[0], fni_ref[0], (((1,), (1,)), ((), ())),
                              preferred_element_type=jnp.float32)
    out_ref[0, 0] += jnp.sum(jnp.where(d2h <= 0.5, 1.0 - cos, 0.0))


def kernel(features, points, proj_pts, invis_idx, height, width):
    del invis_idx
    B, N, D = features.shape
    nr = N // _RB

    fn = pl.pallas_call(
        _normalize_body,
        grid=(B,),
        in_specs=[pl.BlockSpec((1, N, D), lambda b: (b, 0, 0))],
        out_specs=pl.BlockSpec((1, N, D), lambda b: (b, 0, 0)),
        out_shape=jax.ShapeDtypeStruct((B, N, D), jnp.bfloat16),
    )(features)

    factor = jnp.array([(width - 1.0) / 2.0, (height - 1.0) / 2.0],
                       jnp.float32)
    pts = (points + 1.0) * factor
    ppts = (proj_pts + 1.0) * factor
    row = jax.lax.broadcasted_iota(jnp.int32, (N, N), 0)
    col = jax.lax.broadcasted_iota(jnp.int32, (N, N), 1)
    tblk = jnp.where(row <= col, 0.0, 1e30).astype(jnp.float32)

    out = pl.pallas_call(
        _match_body,
        grid=(nr, B, B),
        in_specs=[
            pl.BlockSpec((1, _RB, D), lambda r, i, j: (j, r, 0)),
            pl.BlockSpec((1, N, D), lambda r, i, j: (i, 0, 0)),
            pl.BlockSpec((1, _RB, 2), lambda r, i, j: (i, r, 0)),
            pl.BlockSpec((1, 1, N, 2), lambda r, i, j: (i, j, 0, 0)),
            pl.BlockSpec((_RB, N), lambda r, i, j: (r, 0)),
        ],
        out_specs=pl.BlockSpec((1, 1), lambda r, i, j: (0, 0),
                               memory_space=pltpu.SMEM),
        out_shape=jax.ShapeDtypeStruct((1, 1), jnp.float32),
        compiler_params=pltpu.CompilerParams(
            dimension_semantics=("arbitrary", "arbitrary", "arbitrary")),
    )(fn, fn, pts, ppts, tblk)
    return out[0, 0]
